# baseline (speedup 1.0000x reference)
import functools

import jax
import jax.numpy as jnp
from jax import lax
from jax.experimental import pallas as pl
from jax.experimental.pallas import tpu as pltpu
from jax.experimental.pallas import tpu_sc as plsc

N = 10000
E = 640000
G = 128
D = 101
DP = 128
D2 = 200
D2P = 208
NC = 2
NS = 16
NW = NC * NS
EPW = E // NW
CH = 80
NCHUNK = EPW // CH
RPW = 624
RBLK = 2000


def _edge_body(h_hbm, zero_hbm, src_hbm, dst_hbm, attr_hbm, out_hbm,
               sidx0, sidx1, didx0, didx1, attr0, attr1, rows0, rows1,
               table, sem_i0, sem_i1, sem_g0, sem_g1):
    c = lax.axis_index("c")
    s = lax.axis_index("s")
    row0 = s * RPW

    @pl.when(c == 0)
    def _():
        pltpu.sync_copy(h_hbm.at[pl.ds(row0, RPW)], table.at[pl.ds(row0, RPW)])

        @pl.when(s == NS - 1)
        def _():
            pltpu.sync_copy(h_hbm.at[pl.ds(NS * RPW, N - NS * RPW)],
                            table.at[pl.ds(NS * RPW, N - NS * RPW)])

    @pl.when(c != 0)
    def _():
        pltpu.sync_copy(zero_hbm.at[pl.ds(row0, RPW)],
                        table.at[pl.ds(row0, RPW)])

        @pl.when(s == NS - 1)
        def _():
            pltpu.sync_copy(zero_hbm.at[pl.ds(NS * RPW, N - NS * RPW)],
                            table.at[pl.ds(NS * RPW, N - NS * RPW)])

    z16 = jnp.zeros((16,), jnp.float32)
    attr0[pl.ds(CH * D, 16)] = z16
    attr0[pl.ds(CH * D + 16, 16)] = z16
    attr1[pl.ds(CH * D, 16)] = z16
    attr1[pl.ds(CH * D + 16, 16)] = z16
    plsc.subcore_barrier()

    e0 = (c * NS + s) * EPW
    bufs = ((sidx0, didx0, attr0, rows0, sem_i0, sem_g0),
            (sidx1, didx1, attr1, rows1, sem_i1, sem_g1))

    def issue_in(g, b):
        eb = e0 + g * CH
        si, di, at, _, smi, _ = bufs[b]
        pltpu.async_copy(src_hbm.at[pl.ds(eb, CH)], si, smi)
        pltpu.async_copy(dst_hbm.at[pl.ds(eb, CH)], di, smi)
        pltpu.async_copy(attr_hbm.at[pl.ds(eb * D, CH * D)],
                         at.at[pl.ds(0, CH * D)], smi)

    def wait_in(b):
        si, di, at, _, smi, _ = bufs[b]
        pltpu.make_async_copy(src_hbm.at[pl.ds(0, CH)], si, smi).wait()
        pltpu.make_async_copy(dst_hbm.at[pl.ds(0, CH)], di, smi).wait()
        pltpu.make_async_copy(attr_hbm.at[pl.ds(0, CH * D)],
                              at.at[pl.ds(0, CH * D)], smi).wait()

    def issue_gather(b):
        si, _, _, ro, _, smg = bufs[b]
        pltpu.async_copy(h_hbm.at[si], ro, smg)

    def wait_gather(b):
        si, _, _, ro, _, smg = bufs[b]
        pltpu.make_async_copy(h_hbm.at[si], ro, smg).wait()

    issue_in(0, 0)
    issue_in(1, 1)
    wait_in(0)
    issue_gather(0)

    def step(g, b):
        si, di, at, ro, smi, smg = bufs[b]
        wait_gather(b)

        @plsc.parallel_loop(0, CH, unroll=8)
        def _(e):
            for j in range(7):
                a = at[pl.ds(e * D + j * 16, 16)]
                gv = ro[e, pl.ds(j * 16, 16)]
                ro[e, pl.ds(j * 16, 16)] = jnp.maximum(gv + a, 0.0)

        pltpu.sync_copy(ro, table.at[di], add=True)

        @pl.when(g + 1 < NCHUNK)
        def _():
            wait_in(1 - b)
            issue_gather(1 - b)

        @pl.when(g + 2 < NCHUNK)
        def _():
            issue_in(g + 2, b)

    def outer(it, carry):
        step(2 * it, 0)
        step(2 * it + 1, 1)
        return carry

    lax.fori_loop(0, NCHUNK // 2, outer, 0)
    plsc.subcore_barrier()
    pltpu.sync_copy(table.at[pl.ds(row0, RPW)],
                    out_hbm.at[c, pl.ds(row0, RPW)])

    @pl.when(s == NS - 1)
    def _():
        pltpu.sync_copy(table.at[pl.ds(NS * RPW, N - NS * RPW)],
                        out_hbm.at[c, pl.ds(NS * RPW, N - NS * RPW)])


_edge_call = functools.partial(
    pl.kernel,
    mesh=plsc.VectorSubcoreMesh(core_axis_name="c", subcore_axis_name="s"),
    out_type=jax.ShapeDtypeStruct((NC, N, DP), jnp.float32),
    scratch_types=[
        pltpu.VMEM((CH,), jnp.int32),
        pltpu.VMEM((CH,), jnp.int32),
        pltpu.VMEM((CH,), jnp.int32),
        pltpu.VMEM((CH,), jnp.int32),
        pltpu.VMEM((CH * D + 32,), jnp.float32),
        pltpu.VMEM((CH * D + 32,), jnp.float32),
        pltpu.VMEM((CH, DP), jnp.float32),
        pltpu.VMEM((CH, DP), jnp.float32),
        pltpu.VMEM_SHARED((N, DP), jnp.float32),
        pltpu.SemaphoreType.DMA,
        pltpu.SemaphoreType.DMA,
        pltpu.SemaphoreType.DMA,
        pltpu.SemaphoreType.DMA,
    ],
)(_edge_body)


def _mm(a, b):
    return lax.dot_general(a, b, (((1,), (0,)), ((), ())),
                           preferred_element_type=jnp.float32)


def _mlp1_body(p_ref, W1_ref, b1_ref, W2_ref, b2_ref, out_ref):
    t = p_ref[0] + p_ref[1]
    y = jnp.maximum(_mm(t, W1_ref[...]) + b1_ref[...], 0.0)
    out_ref[...] = jnp.maximum(_mm(y, W2_ref[...]) + b2_ref[...], 0.0)


def _mlp2_body(p_ref, batch_ref, W3_ref, b3_ref, W4_ref, b4_ref, out_ref):
    i = pl.program_id(0)
    t = p_ref[0] + p_ref[1]
    y = jnp.maximum(_mm(t, W3_ref[...]) + b3_ref[...], 0.0)
    h2 = jnp.maximum(_mm(y, W4_ref[...]) + b4_ref[...], 0.0)
    seg = lax.broadcasted_iota(jnp.int32, (RBLK, G), 1)
    onehot = (batch_ref[...] == seg).astype(jnp.float32)
    pooled = lax.dot_general(onehot, h2, (((0,), (0,)), ((), ())),
                             preferred_element_type=jnp.float32,
                             precision=lax.Precision.HIGHEST)

    @pl.when(i == 0)
    def _():
        out_ref[...] = pooled

    @pl.when(i != 0)
    def _():
        out_ref[...] += pooled


def _head_body(p_ref, W5_ref, b5_ref, W6_ref, b6_ref, out_ref):
    y = jnp.maximum(_mm(p_ref[...], W5_ref[...]) + b5_ref[...], 0.0)
    out_ref[...] = jnp.abs(_mm(y, W6_ref[...]) + b6_ref[...])


def _pad2(w, r, c):
    return jnp.pad(w, ((0, r - w.shape[0]), (0, c - w.shape[1])))


def _pad1(b, c):
    return jnp.pad(b, (0, c - b.shape[0])).reshape(1, c)


def kernel(x, edge_index, edge_type, edge_attr, batch,
           W1, b1, W2, b2, W3, b3, W4, b4, W5, b5, W6, b6):
    del edge_type
    x_pad = jnp.pad(x, ((0, 0), (0, DP - D)))
    zeros = jnp.zeros((N, DP), jnp.float32)
    src = edge_index[0]
    dst = edge_index[1]
    attr_flat = edge_attr.reshape(-1)
    batch2d = batch.reshape(N, 1)

    W1p, b1p = _pad2(W1, DP, DP), _pad1(b1, DP)
    W2p, b2p = _pad2(W2, DP, DP), _pad1(b2, DP)
    W3p, b3p = _pad2(W3, DP, DP), _pad1(b3, DP)
    W4p, b4p = _pad2(W4, DP, D2P), _pad1(b4, D2P)
    W5p, b5p = _pad2(W5, D2P, 64), _pad1(b5, 64)
    W6p, b6p = _pad2(W6, 64, 128), _pad1(b6, 128)

    p1 = _edge_call(x_pad, zeros, src, dst, attr_flat)

    nblk = N // RBLK
    h1 = pl.pallas_call(
        _mlp1_body,
        grid=(nblk,),
        in_specs=[
            pl.BlockSpec((NC, RBLK, DP), lambda i: (0, i, 0)),
            pl.BlockSpec((DP, DP), lambda i: (0, 0)),
            pl.BlockSpec((1, DP), lambda i: (0, 0)),
            pl.BlockSpec((DP, DP), lambda i: (0, 0)),
            pl.BlockSpec((1, DP), lambda i: (0, 0)),
        ],
        out_specs=pl.BlockSpec((RBLK, DP), lambda i: (i, 0)),
        out_shape=jax.ShapeDtypeStruct((N, DP), jnp.float32),
    )(p1, W1p, b1p, W2p, b2p)

    p2 = _edge_call(h1, zeros, src, dst, attr_flat)

    pooled = pl.pallas_call(
        _mlp2_body,
        grid=(nblk,),
        in_specs=[
            pl.BlockSpec((NC, RBLK, DP), lambda i: (0, i, 0)),
            pl.BlockSpec((RBLK, 1), lambda i: (i, 0)),
            pl.BlockSpec((DP, DP), lambda i: (0, 0)),
            pl.BlockSpec((1, DP), lambda i: (0, 0)),
            pl.BlockSpec((DP, D2P), lambda i: (0, 0)),
            pl.BlockSpec((1, D2P), lambda i: (0, 0)),
        ],
        out_specs=pl.BlockSpec((G, D2P), lambda i: (0, 0)),
        out_shape=jax.ShapeDtypeStruct((G, D2P), jnp.float32),
    )(p2, batch2d, W3p, b3p, W4p, b4p)

    out = pl.pallas_call(
        _head_body,
        out_shape=jax.ShapeDtypeStruct((G, 128), jnp.float32),
    )(pooled, W5p, b5p, W6p, b6p)
    return out[:, :1]


# speedup vs baseline: 1.1025x; 1.1025x over previous
import functools

import jax
import jax.numpy as jnp
from jax import lax
from jax.experimental import pallas as pl
from jax.experimental.pallas import tpu as pltpu
from jax.experimental.pallas import tpu_sc as plsc

N = 10000
E = 640000
G = 128
D = 101
DP = 128
D2 = 200
D2P = 208
NC = 2
NS = 16
NW = NC * NS
EPW = E // NW
CH = 80
NCHUNK = EPW // CH
RPW = 624
RBLK = 2000


def _edge_body(h_hbm, zero_hbm, src_hbm, dst_hbm, attr_hbm, out_hbm,
               sidx0, sidx1, didx0, didx1, attr0, attr1, rows0, rows1,
               table, sem_i0, sem_i1, sem_g0, sem_g1):
    c = lax.axis_index("c")
    s = lax.axis_index("s")
    row0 = s * RPW

    @pl.when(c == 0)
    def _():
        pltpu.sync_copy(h_hbm.at[pl.ds(row0, RPW)], table.at[pl.ds(row0, RPW)])

        @pl.when(s == NS - 1)
        def _():
            pltpu.sync_copy(h_hbm.at[pl.ds(NS * RPW, N - NS * RPW)],
                            table.at[pl.ds(NS * RPW, N - NS * RPW)])

    @pl.when(c != 0)
    def _():
        pltpu.sync_copy(zero_hbm.at[pl.ds(row0, RPW)],
                        table.at[pl.ds(row0, RPW)])

        @pl.when(s == NS - 1)
        def _():
            pltpu.sync_copy(zero_hbm.at[pl.ds(NS * RPW, N - NS * RPW)],
                            table.at[pl.ds(NS * RPW, N - NS * RPW)])

    plsc.subcore_barrier()

    e0 = (c * NS + s) * EPW
    bufs = ((sidx0, didx0, attr0, rows0, sem_i0, sem_g0),
            (sidx1, didx1, attr1, rows1, sem_i1, sem_g1))

    def issue_in(g, b):
        eb = e0 + g * CH
        si, di, at, _, smi, _ = bufs[b]
        pltpu.async_copy(src_hbm.at[pl.ds(eb, CH)], si, smi)
        pltpu.async_copy(dst_hbm.at[pl.ds(eb, CH)], di, smi)
        pltpu.async_copy(attr_hbm.at[pl.ds(eb, CH)], at, smi)

    def wait_in(b):
        si, di, at, _, smi, _ = bufs[b]
        pltpu.make_async_copy(src_hbm.at[pl.ds(0, CH)], si, smi).wait()
        pltpu.make_async_copy(dst_hbm.at[pl.ds(0, CH)], di, smi).wait()
        pltpu.make_async_copy(attr_hbm.at[pl.ds(0, CH)], at, smi).wait()

    def issue_gather(b):
        si, _, _, ro, _, smg = bufs[b]
        pltpu.async_copy(h_hbm.at[si], ro, smg)

    def wait_gather(b):
        si, _, _, ro, _, smg = bufs[b]
        pltpu.make_async_copy(h_hbm.at[si], ro, smg).wait()

    issue_in(0, 0)
    issue_in(1, 1)
    wait_in(0)
    issue_gather(0)

    def step(g, b):
        si, di, at, ro, smi, smg = bufs[b]
        wait_gather(b)

        @plsc.parallel_loop(0, CH, unroll=8)
        def _(e):
            for j in range(7):
                a = at[e, pl.ds(j * 16, 16)]
                gv = ro[e, pl.ds(j * 16, 16)]
                ro[e, pl.ds(j * 16, 16)] = jnp.maximum(gv + a, 0.0)

        pltpu.sync_copy(ro, table.at[di], add=True)

        @pl.when(g + 1 < NCHUNK)
        def _():
            wait_in(1 - b)
            issue_gather(1 - b)

        @pl.when(g + 2 < NCHUNK)
        def _():
            issue_in(g + 2, b)

    def outer(it, carry):
        step(2 * it, 0)
        step(2 * it + 1, 1)
        return carry

    lax.fori_loop(0, NCHUNK // 2, outer, 0)
    plsc.subcore_barrier()
    pltpu.sync_copy(table.at[pl.ds(row0, RPW)],
                    out_hbm.at[c, pl.ds(row0, RPW)])

    @pl.when(s == NS - 1)
    def _():
        pltpu.sync_copy(table.at[pl.ds(NS * RPW, N - NS * RPW)],
                        out_hbm.at[c, pl.ds(NS * RPW, N - NS * RPW)])


_edge_call = functools.partial(
    pl.kernel,
    mesh=plsc.VectorSubcoreMesh(core_axis_name="c", subcore_axis_name="s"),
    out_type=jax.ShapeDtypeStruct((NC, N, DP), jnp.float32),
    scratch_types=[
        pltpu.VMEM((CH,), jnp.int32),
        pltpu.VMEM((CH,), jnp.int32),
        pltpu.VMEM((CH,), jnp.int32),
        pltpu.VMEM((CH,), jnp.int32),
        pltpu.VMEM((CH, DP), jnp.float32),
        pltpu.VMEM((CH, DP), jnp.float32),
        pltpu.VMEM((CH, DP), jnp.float32),
        pltpu.VMEM((CH, DP), jnp.float32),
        pltpu.VMEM_SHARED((N, DP), jnp.float32),
        pltpu.SemaphoreType.DMA,
        pltpu.SemaphoreType.DMA,
        pltpu.SemaphoreType.DMA,
        pltpu.SemaphoreType.DMA,
    ],
)(_edge_body)


def _mm(a, b):
    return lax.dot_general(a, b, (((1,), (0,)), ((), ())),
                           preferred_element_type=jnp.float32)


def _mlp1_body(p_ref, W1_ref, b1_ref, W2_ref, b2_ref, out_ref):
    t = p_ref[0] + p_ref[1]
    y = jnp.maximum(_mm(t, W1_ref[...]) + b1_ref[...], 0.0)
    out_ref[...] = jnp.maximum(_mm(y, W2_ref[...]) + b2_ref[...], 0.0)


def _mlp2_body(p_ref, batch_ref, W3_ref, b3_ref, W4_ref, b4_ref, out_ref):
    i = pl.program_id(0)
    t = p_ref[0] + p_ref[1]
    y = jnp.maximum(_mm(t, W3_ref[...]) + b3_ref[...], 0.0)
    h2 = jnp.maximum(_mm(y, W4_ref[...]) + b4_ref[...], 0.0)
    seg = lax.broadcasted_iota(jnp.int32, (RBLK, G), 1)
    onehot = (batch_ref[...] == seg).astype(jnp.float32)
    pooled = lax.dot_general(onehot, h2, (((0,), (0,)), ((), ())),
                             preferred_element_type=jnp.float32,
                             precision=lax.Precision.HIGHEST)

    @pl.when(i == 0)
    def _():
        out_ref[...] = pooled

    @pl.when(i != 0)
    def _():
        out_ref[...] += pooled


def _head_body(p_ref, W5_ref, b5_ref, W6_ref, b6_ref, out_ref):
    y = jnp.maximum(_mm(p_ref[...], W5_ref[...]) + b5_ref[...], 0.0)
    out_ref[...] = jnp.abs(_mm(y, W6_ref[...]) + b6_ref[...])


def _pad2(w, r, c):
    return jnp.pad(w, ((0, r - w.shape[0]), (0, c - w.shape[1])))


def _pad1(b, c):
    return jnp.pad(b, (0, c - b.shape[0])).reshape(1, c)


def kernel(x, edge_index, edge_type, edge_attr, batch,
           W1, b1, W2, b2, W3, b3, W4, b4, W5, b5, W6, b6):
    del edge_type
    x_pad = jnp.pad(x, ((0, 0), (0, DP - D)))
    zeros = jnp.zeros((N, DP), jnp.float32)
    src = edge_index[0]
    dst = edge_index[1]
    attr_pad = jnp.pad(edge_attr, ((0, 0), (0, DP - D)))
    batch2d = batch.reshape(N, 1)

    W1p, b1p = _pad2(W1, DP, DP), _pad1(b1, DP)
    W2p, b2p = _pad2(W2, DP, DP), _pad1(b2, DP)
    W3p, b3p = _pad2(W3, DP, DP), _pad1(b3, DP)
    W4p, b4p = _pad2(W4, DP, D2P), _pad1(b4, D2P)
    W5p, b5p = _pad2(W5, D2P, 64), _pad1(b5, 64)
    W6p, b6p = _pad2(W6, 64, 128), _pad1(b6, 128)

    p1 = _edge_call(x_pad, zeros, src, dst, attr_pad)

    nblk = N // RBLK
    h1 = pl.pallas_call(
        _mlp1_body,
        grid=(nblk,),
        in_specs=[
            pl.BlockSpec((NC, RBLK, DP), lambda i: (0, i, 0)),
            pl.BlockSpec((DP, DP), lambda i: (0, 0)),
            pl.BlockSpec((1, DP), lambda i: (0, 0)),
            pl.BlockSpec((DP, DP), lambda i: (0, 0)),
            pl.BlockSpec((1, DP), lambda i: (0, 0)),
        ],
        out_specs=pl.BlockSpec((RBLK, DP), lambda i: (i, 0)),
        out_shape=jax.ShapeDtypeStruct((N, DP), jnp.float32),
    )(p1, W1p, b1p, W2p, b2p)

    p2 = _edge_call(h1, zeros, src, dst, attr_pad)

    pooled = pl.pallas_call(
        _mlp2_body,
        grid=(nblk,),
        in_specs=[
            pl.BlockSpec((NC, RBLK, DP), lambda i: (0, i, 0)),
            pl.BlockSpec((RBLK, 1), lambda i: (i, 0)),
            pl.BlockSpec((DP, DP), lambda i: (0, 0)),
            pl.BlockSpec((1, DP), lambda i: (0, 0)),
            pl.BlockSpec((DP, D2P), lambda i: (0, 0)),
            pl.BlockSpec((1, D2P), lambda i: (0, 0)),
        ],
        out_specs=pl.BlockSpec((G, D2P), lambda i: (0, 0)),
        out_shape=jax.ShapeDtypeStruct((G, D2P), jnp.float32),
    )(p2, batch2d, W3p, b3p, W4p, b4p)

    out = pl.pallas_call(
        _head_body,
        out_shape=jax.ShapeDtypeStruct((G, 128), jnp.float32),
    )(pooled, W5p, b5p, W6p, b6p)
    return out[:, :1]


# speedup vs baseline: 1.2476x; 1.1316x over previous
import functools

import jax
import jax.numpy as jnp
from jax import lax
from jax.experimental import pallas as pl
from jax.experimental.pallas import tpu as pltpu
from jax.experimental.pallas import tpu_sc as plsc

N = 10000
E = 640000
G = 128
D = 101
DP = 128
D2 = 200
D2P = 208
NC = 2
NS = 16
NW = NC * NS
EPW = E // NW
CH = 80
NCHUNK = EPW // CH
RPW = 624
RBLK = 2000


def _edge_body(h_hbm, zero_hbm, src_hbm, dst_hbm, attr_hbm, out_hbm,
               sidx0, sidx1, didx0, didx1, attr0, attr1, rows0, rows1,
               table, sem_i0, sem_i1, sem_g0, sem_g1,
               sem_s0, sem_s1, sem_d0, sem_d1):
    c = lax.axis_index("c")
    s = lax.axis_index("s")
    row0 = s * RPW

    @pl.when(c == 0)
    def _():
        pltpu.sync_copy(h_hbm.at[pl.ds(row0, RPW)], table.at[pl.ds(row0, RPW)])

        @pl.when(s == NS - 1)
        def _():
            pltpu.sync_copy(h_hbm.at[pl.ds(NS * RPW, N - NS * RPW)],
                            table.at[pl.ds(NS * RPW, N - NS * RPW)])

    @pl.when(c != 0)
    def _():
        pltpu.sync_copy(zero_hbm.at[pl.ds(row0, RPW)],
                        table.at[pl.ds(row0, RPW)])

        @pl.when(s == NS - 1)
        def _():
            pltpu.sync_copy(zero_hbm.at[pl.ds(NS * RPW, N - NS * RPW)],
                            table.at[pl.ds(NS * RPW, N - NS * RPW)])

    plsc.subcore_barrier()

    e0 = (c * NS + s) * EPW
    bufs = ((sidx0, didx0, attr0, rows0, sem_i0, sem_g0, sem_s0, sem_d0),
            (sidx1, didx1, attr1, rows1, sem_i1, sem_g1, sem_s1, sem_d1))

    def issue_in(g, b):
        eb = e0 + g * CH
        si, _, at, _, smi, _, _, _ = bufs[b]
        pltpu.async_copy(src_hbm.at[pl.ds(eb, CH)], si, smi)
        pltpu.async_copy(attr_hbm.at[pl.ds(eb, CH)], at, smi)

    def wait_in(b):
        si, _, at, _, smi, _, _, _ = bufs[b]
        pltpu.make_async_copy(src_hbm.at[pl.ds(0, CH)], si, smi).wait()
        pltpu.make_async_copy(attr_hbm.at[pl.ds(0, CH)], at, smi).wait()

    def issue_didx(g, b):
        eb = e0 + g * CH
        di, smd = bufs[b][1], bufs[b][7]
        pltpu.async_copy(dst_hbm.at[pl.ds(eb, CH)], di, smd)

    def wait_didx(b):
        di, smd = bufs[b][1], bufs[b][7]
        pltpu.make_async_copy(dst_hbm.at[pl.ds(0, CH)], di, smd).wait()

    def issue_gather(b):
        si, _, _, ro, _, smg, _, _ = bufs[b]
        pltpu.async_copy(h_hbm.at[si], ro, smg)

    def wait_gather(b):
        si, _, _, ro, _, smg, _, _ = bufs[b]
        pltpu.make_async_copy(h_hbm.at[si], ro, smg).wait()

    def wait_scatter(b):
        di, ro, sms = bufs[b][1], bufs[b][3], bufs[b][6]
        pltpu.make_async_copy(ro, table.at[di], sms).wait()

    issue_in(0, 0)
    issue_in(1, 1)
    issue_didx(0, 0)
    wait_in(0)
    issue_gather(0)

    def step(g, b):
        si, di, at, ro, smi, smg, sms, smd = bufs[b]
        wait_gather(b)

        @plsc.parallel_loop(0, CH, unroll=8)
        def _(e):
            for j in range(7):
                a = at[e, pl.ds(j * 16, 16)]
                gv = ro[e, pl.ds(j * 16, 16)]
                ro[e, pl.ds(j * 16, 16)] = jnp.maximum(gv + a, 0.0)

        wait_didx(b)
        pltpu.async_copy(ro, table.at[di], sms, add=True)

        @pl.when(g + 1 < NCHUNK)
        def _():
            @pl.when(g >= 1)
            def _():
                wait_scatter(1 - b)

            wait_in(1 - b)
            issue_gather(1 - b)
            issue_didx(g + 1, 1 - b)

        @pl.when(g + 2 < NCHUNK)
        def _():
            issue_in(g + 2, b)

    def outer(it, carry):
        step(2 * it, 0)
        step(2 * it + 1, 1)
        return carry

    lax.fori_loop(0, NCHUNK // 2, outer, 0)
    wait_scatter(0)
    wait_scatter(1)
    plsc.subcore_barrier()
    pltpu.sync_copy(table.at[pl.ds(row0, RPW)],
                    out_hbm.at[c, pl.ds(row0, RPW)])

    @pl.when(s == NS - 1)
    def _():
        pltpu.sync_copy(table.at[pl.ds(NS * RPW, N - NS * RPW)],
                        out_hbm.at[c, pl.ds(NS * RPW, N - NS * RPW)])


_edge_call = functools.partial(
    pl.kernel,
    mesh=plsc.VectorSubcoreMesh(core_axis_name="c", subcore_axis_name="s"),
    out_type=jax.ShapeDtypeStruct((NC, N, DP), jnp.float32),
    scratch_types=[
        pltpu.VMEM((CH,), jnp.int32),
        pltpu.VMEM((CH,), jnp.int32),
        pltpu.VMEM((CH,), jnp.int32),
        pltpu.VMEM((CH,), jnp.int32),
        pltpu.VMEM((CH, DP), jnp.float32),
        pltpu.VMEM((CH, DP), jnp.float32),
        pltpu.VMEM((CH, DP), jnp.float32),
        pltpu.VMEM((CH, DP), jnp.float32),
        pltpu.VMEM_SHARED((N, DP), jnp.float32),
        pltpu.SemaphoreType.DMA,
        pltpu.SemaphoreType.DMA,
        pltpu.SemaphoreType.DMA,
        pltpu.SemaphoreType.DMA,
        pltpu.SemaphoreType.DMA,
        pltpu.SemaphoreType.DMA,
        pltpu.SemaphoreType.DMA,
        pltpu.SemaphoreType.DMA,
    ],
)(_edge_body)


def _mm(a, b):
    return lax.dot_general(a, b, (((1,), (0,)), ((), ())),
                           preferred_element_type=jnp.float32)


def _mlp1_body(p_ref, W1_ref, b1_ref, W2_ref, b2_ref, out_ref):
    t = p_ref[0] + p_ref[1]
    y = jnp.maximum(_mm(t, W1_ref[...]) + b1_ref[...], 0.0)
    out_ref[...] = jnp.maximum(_mm(y, W2_ref[...]) + b2_ref[...], 0.0)


def _mlp2_body(p_ref, batch_ref, W3_ref, b3_ref, W4_ref, b4_ref, out_ref):
    i = pl.program_id(0)
    t = p_ref[0] + p_ref[1]
    y = jnp.maximum(_mm(t, W3_ref[...]) + b3_ref[...], 0.0)
    h2 = jnp.maximum(_mm(y, W4_ref[...]) + b4_ref[...], 0.0)
    seg = lax.broadcasted_iota(jnp.int32, (RBLK, G), 1)
    onehot = (batch_ref[...] == seg).astype(jnp.float32)
    pooled = lax.dot_general(onehot, h2, (((0,), (0,)), ((), ())),
                             preferred_element_type=jnp.float32,
                             precision=lax.Precision.HIGHEST)

    @pl.when(i == 0)
    def _():
        out_ref[...] = pooled

    @pl.when(i != 0)
    def _():
        out_ref[...] += pooled


def _head_body(p_ref, W5_ref, b5_ref, W6_ref, b6_ref, out_ref):
    y = jnp.maximum(_mm(p_ref[...], W5_ref[...]) + b5_ref[...], 0.0)
    out_ref[...] = jnp.abs(_mm(y, W6_ref[...]) + b6_ref[...])


def _pad2(w, r, c):
    return jnp.pad(w, ((0, r - w.shape[0]), (0, c - w.shape[1])))


def _pad1(b, c):
    return jnp.pad(b, (0, c - b.shape[0])).reshape(1, c)


def kernel(x, edge_index, edge_type, edge_attr, batch,
           W1, b1, W2, b2, W3, b3, W4, b4, W5, b5, W6, b6):
    del edge_type
    x_pad = jnp.pad(x, ((0, 0), (0, DP - D)))
    zeros = jnp.zeros((N, DP), jnp.float32)
    src = edge_index[0]
    dst = edge_index[1]
    attr_pad = jnp.pad(edge_attr, ((0, 0), (0, DP - D)))
    batch2d = batch.reshape(N, 1)

    W1p, b1p = _pad2(W1, DP, DP), _pad1(b1, DP)
    W2p, b2p = _pad2(W2, DP, DP), _pad1(b2, DP)
    W3p, b3p = _pad2(W3, DP, DP), _pad1(b3, DP)
    W4p, b4p = _pad2(W4, DP, D2P), _pad1(b4, D2P)
    W5p, b5p = _pad2(W5, D2P, 64), _pad1(b5, 64)
    W6p, b6p = _pad2(W6, 64, 128), _pad1(b6, 128)

    p1 = _edge_call(x_pad, zeros, src, dst, attr_pad)

    nblk = N // RBLK
    h1 = pl.pallas_call(
        _mlp1_body,
        grid=(nblk,),
        in_specs=[
            pl.BlockSpec((NC, RBLK, DP), lambda i: (0, i, 0)),
            pl.BlockSpec((DP, DP), lambda i: (0, 0)),
            pl.BlockSpec((1, DP), lambda i: (0, 0)),
            pl.BlockSpec((DP, DP), lambda i: (0, 0)),
            pl.BlockSpec((1, DP), lambda i: (0, 0)),
        ],
        out_specs=pl.BlockSpec((RBLK, DP), lambda i: (i, 0)),
        out_shape=jax.ShapeDtypeStruct((N, DP), jnp.float32),
    )(p1, W1p, b1p, W2p, b2p)

    p2 = _edge_call(h1, zeros, src, dst, attr_pad)

    pooled = pl.pallas_call(
        _mlp2_body,
        grid=(nblk,),
        in_specs=[
            pl.BlockSpec((NC, RBLK, DP), lambda i: (0, i, 0)),
            pl.BlockSpec((RBLK, 1), lambda i: (i, 0)),
            pl.BlockSpec((DP, DP), lambda i: (0, 0)),
            pl.BlockSpec((1, DP), lambda i: (0, 0)),
            pl.BlockSpec((DP, D2P), lambda i: (0, 0)),
            pl.BlockSpec((1, D2P), lambda i: (0, 0)),
        ],
        out_specs=pl.BlockSpec((G, D2P), lambda i: (0, 0)),
        out_shape=jax.ShapeDtypeStruct((G, D2P), jnp.float32),
    )(p2, batch2d, W3p, b3p, W4p, b4p)

    out = pl.pallas_call(
        _head_body,
        out_shape=jax.ShapeDtypeStruct((G, 128), jnp.float32),
    )(pooled, W5p, b5p, W6p, b6p)
    return out[:, :1]
